# Initial kernel scaffold; baseline (speedup 1.0000x reference)
import jax
import jax.numpy as jnp
from jax.experimental import pallas as pl


def kernel(rpn_cls_prob_reshape, rpn_bbox_pred, im_info):
    raise NotImplementedError("write your pallas kernel here")


# speedup vs baseline: 14.4873x; 14.4873x over previous
import functools
import math

import jax
import jax.numpy as jnp
import numpy as np
from jax import lax
from jax.experimental import pallas as pl
from jax.experimental.pallas import tpu as pltpu

_FEAT_STRIDE = 16
_ANCHOR_SCALES = (4.0, 8.0, 16.0, 32.0)
_ANCHOR_RATIOS = (0.5, 1.0, 2.0)
_PRE_NMS_TOP_N = 6000
_POST_NMS_TOP_N = 300
_NMS_THRESH = 0.7
_MIN_SIZE = 16.0
_NEG = -1e9
_PAD_SCORE = -3.0e38
_LANES = 128


def _base_anchors(base_size=16):
    ratios = np.array(_ANCHOR_RATIOS)
    scales = np.array(_ANCHOR_SCALES)
    base = np.array([1.0, 1.0, float(base_size), float(base_size)]) - 1.0
    w = base[2] - base[0] + 1.0
    h = base[3] - base[1] + 1.0
    cx = base[0] + 0.5 * (w - 1.0)
    cy = base[1] + 0.5 * (h - 1.0)

    def make(ws, hs, cx, cy):
        hw = 0.5 * (ws - 1.0)
        hh = 0.5 * (hs - 1.0)
        return np.stack([cx - hw, cy - hh, cx + hw, cy + hh], axis=1)

    size_ratios = w * h / ratios
    ws = np.round(np.sqrt(size_ratios))
    hs = np.round(ws * ratios)
    ratio_anchors = make(ws, hs, cx, cy)
    out = []
    for ra in ratio_anchors:
        w2 = ra[2] - ra[0] + 1.0
        h2 = ra[3] - ra[1] + 1.0
        cx2 = ra[0] + 0.5 * (w2 - 1.0)
        cy2 = ra[1] + 0.5 * (h2 - 1.0)
        out.append(make(w2 * scales, h2 * scales, cx2, cy2))
    return np.vstack(out).astype(np.float32)


@functools.lru_cache(maxsize=None)
def _anchor_stats(H, W):
    base = _base_anchors()
    A = base.shape[0]
    sy, sx = np.meshgrid(np.arange(H) * _FEAT_STRIDE,
                         np.arange(W) * _FEAT_STRIDE, indexing="ij")
    shifts = np.stack([sx.ravel(), sy.ravel(), sx.ravel(), sy.ravel()],
                      axis=1).astype(np.float32)
    anchors = (base[None, :, :] + shifts[:, None, :]).reshape(-1, 4)
    widths = anchors[:, 2] - anchors[:, 0] + 1.0
    heights = anchors[:, 3] - anchors[:, 1] + 1.0
    ctr_x = anchors[:, 0] + 0.5 * widths
    ctr_y = anchors[:, 1] + 0.5 * heights
    N = anchors.shape[0]
    R = math.ceil(N / _LANES)
    NP = R * _LANES

    def padr(a):
        return np.pad(a, (0, NP - N)).reshape(R, _LANES).astype(np.float32)

    return N, R, NP, padr(widths), padr(heights), padr(ctr_x), padr(ctr_y)


def _nms_kernel(N, R, NP,
                s_ref, dx_ref, dy_ref, dw_ref, dh_ref,
                aw_ref, ah_ref, acx_ref, acy_ref, im_ref,
                out_ref,
                sw_ref, x1_ref, y1_ref, x2p_ref, y2p_ref, ar_ref,
                x2c_ref, y2c_ref):
    h_im = im_ref[0]
    w_im = im_ref[1]
    scale = im_ref[2]

    flat_iota = (lax.broadcasted_iota(jnp.int32, (R, _LANES), 0) * _LANES
                 + lax.broadcasted_iota(jnp.int32, (R, _LANES), 1))

    aw = aw_ref[...]
    ah = ah_ref[...]
    pcx = dx_ref[...] * aw + acx_ref[...]
    pcy = dy_ref[...] * ah + acy_ref[...]
    pw = jnp.exp(dw_ref[...]) * aw
    ph = jnp.exp(dh_ref[...]) * ah
    x1 = pcx - 0.5 * pw
    y1 = pcy - 0.5 * ph
    x2 = pcx + 0.5 * pw
    y2 = pcy + 0.5 * ph
    x1c = jnp.minimum(jnp.maximum(x1, 0.0), w_im - 1.0)
    y1c = jnp.minimum(jnp.maximum(y1, 0.0), h_im - 1.0)
    x2c = jnp.minimum(jnp.maximum(x2, 0.0), w_im - 1.0)
    y2c = jnp.minimum(jnp.maximum(y2, 0.0), h_im - 1.0)

    ms1 = _MIN_SIZE * scale - 1.0
    keep = ((x2c - x1c) >= ms1) & ((y2c - y1c) >= ms1)
    s = jnp.where(keep, s_ref[...], jnp.float32(_NEG))
    s = jnp.where(flat_iota < N, s, jnp.float32(_PAD_SCORE))

    kbits = lax.bitcast_convert_type(s, jnp.int32)
    key = jnp.where(kbits < 0, kbits ^ jnp.int32(0x7FFFFFFF), kbits)

    K = min(_PRE_NMS_TOP_N, N)
    Kf = jnp.float32(K)

    def cnt(pred):
        return jnp.sum(pred.astype(jnp.float32))

    c_pos = cnt(key >= 0)
    base = jnp.where(c_pos >= Kf, jnp.int32(0), jnp.int32(-2147483648))
    for b in range(30, -1, -1):
        cand = base | jnp.int32(1 << b)
        c = cnt(key >= cand)
        base = jnp.where(c >= Kf, cand, base)
    V = base
    c_gt = cnt(key > V)
    need_eq = Kf - c_gt
    eq = key == V
    lo = jnp.int32(0)
    hi = jnp.int32(NP - 1)
    for _ in range(16):
        mid = (lo + hi) // 2
        c = cnt(eq & (flat_iota <= mid))
        ge = c >= need_eq
        hi = jnp.where(ge, mid, hi)
        lo = jnp.where(ge, lo, mid + 1)
    elig = (key > V) | (eq & (flat_iota <= hi))

    sw_ref[...] = jnp.where(elig, s, jnp.float32(_NEG))
    x1_ref[...] = x1c
    y1_ref[...] = y1c
    x2c_ref[...] = x2c
    y2c_ref[...] = y2c
    x2p = x2c + 1.0
    y2p = y2c + 1.0
    x2p_ref[...] = x2p
    y2p_ref[...] = y2p
    ar_ref[...] = (x2p - x1c) * (y2p - y1c)

    lane_row = lax.broadcasted_iota(jnp.int32, (1, _LANES), 1)
    valid_cut = jnp.float32(_NEG * 0.5)

    def body(i, carry):
        sw = sw_ref[...]
        m = jnp.max(sw)
        idx = jnp.min(jnp.where(sw == m, flat_iota, jnp.int32(NP)))
        r = idx // _LANES
        l = idx - r * _LANES
        lm = lane_row == l

        def ext(ref):
            row = ref[pl.ds(r, 1), :]
            return jnp.sum(jnp.where(lm, row, 0.0))

        x1b = ext(x1_ref)
        y1b = ext(y1_ref)
        x2pb = ext(x2p_ref)
        y2pb = ext(y2p_ref)
        arb = ext(ar_ref)
        x2cb = ext(x2c_ref)
        y2cb = ext(y2c_ref)
        kval = jnp.where(m > valid_cut, jnp.float32(1.0), jnp.float32(0.0))

        xx1 = jnp.maximum(x1b, x1_ref[...])
        yy1 = jnp.maximum(y1b, y1_ref[...])
        xx2 = jnp.minimum(x2pb, x2p_ref[...])
        yy2 = jnp.minimum(y2pb, y2p_ref[...])
        inter = jnp.maximum(xx2 - xx1, 0.0) * jnp.maximum(yy2 - yy1, 0.0)
        iou = inter / (arb + ar_ref[...] - inter + 1e-9)
        sw_ref[...] = jnp.where((iou > _NMS_THRESH) | (flat_iota == idx),
                                jnp.float32(_NEG), sw)

        vals = jnp.zeros((1, _LANES), jnp.float32)
        for j, v in enumerate((x1b, y1b, x2cb, y2cb, m)):
            vals = jnp.where(lane_row == j, v * kval, vals)
        out_ref[pl.ds(i, 1), :] = vals
        return carry

    lax.fori_loop(0, _POST_NMS_TOP_N, body, 0)


def kernel(rpn_cls_prob_reshape, rpn_bbox_pred, im_info):
    H, W = rpn_cls_prob_reshape.shape[-2], rpn_cls_prob_reshape.shape[-1]
    A = _base_anchors().shape[0]
    N, R, NP, aw, ah, acx, acy = _anchor_stats(H, W)

    deltas = jnp.transpose(rpn_bbox_pred, (0, 2, 3, 1)).reshape(-1, 4)
    scores = jnp.transpose(rpn_cls_prob_reshape[:, A:], (0, 2, 3, 1)).ravel()

    def padr(a, val=0.0):
        return jnp.pad(a, (0, NP - N), constant_values=val).reshape(R, _LANES)

    s_in = padr(scores, _PAD_SCORE)
    dx = padr(deltas[:, 0])
    dy = padr(deltas[:, 1])
    dw = padr(deltas[:, 2])
    dh = padr(deltas[:, 3])
    im_sm = im_info.reshape(-1)[:3]

    out_rows = _POST_NMS_TOP_N + (-_POST_NMS_TOP_N) % 8
    vspec = pl.BlockSpec(memory_space=pltpu.VMEM)
    out = pl.pallas_call(
        functools.partial(_nms_kernel, N, R, NP),
        out_shape=jax.ShapeDtypeStruct((out_rows, _LANES), jnp.float32),
        in_specs=[vspec] * 9 + [pl.BlockSpec(memory_space=pltpu.SMEM)],
        out_specs=vspec,
        scratch_shapes=[pltpu.VMEM((R, _LANES), jnp.float32)] * 8,
    )(s_in, dx, dy, dw, dh,
      jnp.asarray(aw), jnp.asarray(ah), jnp.asarray(acx), jnp.asarray(acy),
      im_sm)

    res = out[:_POST_NMS_TOP_N]
    rois = jnp.concatenate(
        [jnp.zeros((_POST_NMS_TOP_N, 1), jnp.float32), res[:, 0:4]], axis=1)
    scores_k = res[:, 4]
    return rois, scores_k
